# baseline (speedup 1.0000x reference)
import functools

import jax
import jax.numpy as jnp
from jax import lax
from jax.experimental import pallas as pl
from jax.experimental.pallas import tpu as pltpu
from jax.experimental.pallas import tpu_sc as plsc

EMBED_DIM = 64
BATCH = 4096
SEQ_LEN = 200

_info = plsc.get_sparse_core_info()
NC, NS = _info.num_cores, _info.num_subcores
NW = NC * NS
BPW = BATCH // NW

NB = 4
NT = 2
NBODY = SEQ_LEN // NB

_mesh = plsc.VectorSubcoreMesh(core_axis_name="c", subcore_axis_name="s")


@functools.partial(
    pl.kernel,
    mesh=_mesh,
    out_type=jax.ShapeDtypeStruct((SEQ_LEN, EMBED_DIM, BATCH), jnp.float32),
    scratch_types=[
        pltpu.VMEM((BPW, SEQ_LEN), jnp.int32),
        pltpu.VMEM((SEQ_LEN, BPW), jnp.int32),
        pltpu.VMEM((NB, BPW, EMBED_DIM), jnp.float32),
        pltpu.VMEM((NT, EMBED_DIM, BPW), jnp.float32),
        pltpu.SemaphoreType.DMA((NB,)),
        pltpu.SemaphoreType.DMA((NT,)),
    ],
    compiler_params=pltpu.CompilerParams(
        use_tc_tiling_on_sc=False, needs_layout_passes=False
    ),
)
def _gather_t(idx_hbm, table_hbm, out_hbm, idx_raw, idx_t, g, t, gsem, ssem):
    wid = lax.axis_index("s") * NC + lax.axis_index("c")
    b0 = wid * BPW
    iota = lax.iota(jnp.int32, 16)

    pltpu.sync_copy(idx_hbm.at[pl.ds(b0, BPW)], idx_raw)

    def idx_body(c, carry):
        col = jnp.zeros((16,), jnp.int32) + c
        for j0 in range(BPW // 16):
            idx_t[c, pl.ds(j0 * 16, 16)] = plsc.load_gather(
                idx_raw, [iota + j0 * 16, col]
            )
        return carry

    lax.fori_loop(0, SEQ_LEN, idx_body, 0)

    def fire_gather(s, slot):
        return pltpu.async_copy(
            table_hbm.at[idx_t.at[s]], g.at[slot], gsem.at[slot]
        )

    for b in range(NB):
        fire_gather(b, b)

    def body(k, carry):
        for b in range(NB):
            s = k * NB + b
            tb = b % NT
            pltpu.make_async_copy(
                table_hbm.at[idx_t.at[s]], g.at[b], gsem.at[b]
            ).wait()

            if b >= NT:
                pltpu.make_async_copy(
                    t.at[tb], out_hbm.at[s - NT, :, pl.ds(b0, BPW)], ssem.at[tb]
                ).wait()
            else:

                @pl.when(k > 0)
                def _():
                    pltpu.make_async_copy(
                        t.at[tb], out_hbm.at[s - NT, :, pl.ds(b0, BPW)], ssem.at[tb]
                    ).wait()

            def tr_body(c, carry, _b=b, _tb=tb):
                col = jnp.zeros((16,), jnp.int32) + c
                for j0 in range(BPW // 16):
                    t[_tb, c, pl.ds(j0 * 16, 16)] = plsc.load_gather(
                        g.at[_b], [iota + j0 * 16, col]
                    )
                return carry

            lax.fori_loop(0, EMBED_DIM, tr_body, 0)

            pltpu.async_copy(t.at[tb], out_hbm.at[s, :, pl.ds(b0, BPW)], ssem.at[tb])

            @pl.when(k < NBODY - 1)
            def _():
                fire_gather(s + NB, b)
        return carry

    lax.fori_loop(0, NBODY, body, 0)

    for b in range(NB - NT, NB):
        tb = b % NT
        s = (NBODY - 1) * NB + b
        pltpu.make_async_copy(
            t.at[tb], out_hbm.at[s, :, pl.ds(b0, BPW)], ssem.at[tb]
        ).wait()


def kernel(input_ids, table):
    out_t = _gather_t(input_ids.astype(jnp.int32), table)
    return jnp.transpose(out_t, (2, 0, 1))


# speedup vs baseline: 1.6524x; 1.6524x over previous
import functools

import jax
import jax.numpy as jnp
from jax import lax
from jax.experimental import pallas as pl
from jax.experimental.pallas import tpu as pltpu
from jax.experimental.pallas import tpu_sc as plsc

EMBED_DIM = 64
BATCH = 4096
SEQ_LEN = 200

_info = plsc.get_sparse_core_info()
NC, NS = _info.num_cores, _info.num_subcores
NW = NC * NS
RPW = BATCH // NW

SPLIT = (0, 104, 200)
NB = 4
NBODY = RPW // NB

_mesh = plsc.VectorSubcoreMesh(core_axis_name="c", subcore_axis_name="s")


@functools.partial(
    pl.kernel,
    mesh=_mesh,
    out_type=jax.ShapeDtypeStruct((BATCH, SEQ_LEN, EMBED_DIM), jnp.float32),
    scratch_types=[
        pltpu.VMEM((RPW, SEQ_LEN), jnp.int32),
        pltpu.VMEM((NB, SEQ_LEN, EMBED_DIM), jnp.float32),
        pltpu.SemaphoreType.DMA((NB,)),
        pltpu.SemaphoreType.DMA((NB,)),
    ],
    compiler_params=pltpu.CompilerParams(use_tc_tiling_on_sc=False),
)
def _gather_rows(idx_hbm, table_hbm, out_hbm, idx_all, rows, gsem, ssem):
    wid = lax.axis_index("s") * NC + lax.axis_index("c")
    base = wid * RPW
    pltpu.sync_copy(idx_hbm.at[pl.ds(base, RPW)], idx_all)

    def body(k, carry):
        gathers = []
        for b in range(NB):
            r = k * NB + b

            @pl.when(k > 0)
            def _():
                pltpu.make_async_copy(rows.at[b], out_hbm.at[base + r], ssem.at[b]).wait()

            for lo, hi in zip(SPLIT[:-1], SPLIT[1:]):
                gathers.append(
                    pltpu.async_copy(
                        table_hbm.at[idx_all.at[r, pl.ds(lo, hi - lo)]],
                        rows.at[b, pl.ds(lo, hi - lo)],
                        gsem.at[b],
                    )
                )
        for b in range(NB):
            r = k * NB + b
            for j in range(len(SPLIT) - 1):
                gathers[b * (len(SPLIT) - 1) + j].wait()
            pltpu.async_copy(rows.at[b], out_hbm.at[base + r], ssem.at[b])
        return carry

    lax.fori_loop(0, NBODY, body, 0)

    for b in range(NB):
        r = (NBODY - 1) * NB + b
        pltpu.make_async_copy(rows.at[b], out_hbm.at[base + r], ssem.at[b]).wait()


def kernel(input_ids, table):
    return _gather_rows(input_ids.astype(jnp.int32), table)
